# Initial kernel scaffold; baseline (speedup 1.0000x reference)
import jax
import jax.numpy as jnp
from jax.experimental import pallas as pl


def kernel(z, edge_index):
    raise NotImplementedError("write your pallas kernel here")


# speedup vs baseline: 1.3385x; 1.3385x over previous
import functools

import jax
import jax.numpy as jnp
from jax import lax
from jax.experimental import pallas as pl
from jax.experimental.pallas import tpu as pltpu
from jax.experimental.pallas import tpu_sc as plsc

E = 320000
N = 10000
D = 128
NC = 2
NS = 16
L = 16
NW = NC * NS
EPW = E // NW
K = 80
NCHUNK = EPW // K
G = K // L
DSTEP = 8


def _sc_body(z_hbm, src_hbm, dst_hbm, out_hbm,
             sidx, didx, sr0, dr0, sr1, dr1, outv, sem0, sem1):
    wid = lax.axis_index("s") * NC + lax.axis_index("c")
    base = wid * EPW

    pltpu.sync_copy(src_hbm.at[pl.ds(base, EPW)], sidx)
    pltpu.sync_copy(dst_hbm.at[pl.ds(base, EPW)], didx)

    srows = (sr0, sr1)
    drows = (dr0, dr1)
    sems = (sem0, sem1)

    def issue(c, b):
        pltpu.async_copy(z_hbm.at[sidx.at[pl.ds(c * K, K)]],
                         srows[b], sems[b])
        pltpu.async_copy(z_hbm.at[didx.at[pl.ds(c * K, K)]],
                         drows[b], sems[b])

    def wait(c, b):
        pltpu.make_async_copy(
            z_hbm.at[sidx.at[pl.ds(c * K, K)]],
            srows[b], sems[b]).wait()
        pltpu.make_async_copy(
            z_hbm.at[didx.at[pl.ds(c * K, K)]],
            drows[b], sems[b]).wait()

    lanes = lax.iota(jnp.int32, L)

    def compute(c, b):
        sref = srows[b]
        dref = drows[b]
        for g in range(G):
            eids = lanes + (g * L)

            def dstep(t, acc):
                for dd in range(DSTEP):
                    d = t * DSTEP + dd
                    dvec = jnp.full((L,), d, dtype=jnp.int32)
                    sv = plsc.load_gather(sref, [eids, dvec])
                    dv = plsc.load_gather(dref, [eids, dvec])
                    acc = acc + sv * dv
                return acc

            acc = lax.fori_loop(0, D // DSTEP, dstep,
                                jnp.zeros((L,), jnp.float32))
            res = 1.0 / (1.0 + jnp.exp(-acc))
            outv[pl.ds(c * K + g * L, L)] = res

    issue(0, 0)
    issue(1, 1)

    def chunk_pair(i, carry):
        for b in range(2):
            c = 2 * i + b
            wait(c, b)
            compute(c, b)

            @pl.when(c + 2 < NCHUNK)
            def _():
                issue(c + 2, b)
        return carry

    lax.fori_loop(0, (NCHUNK - 1) // 2, chunk_pair, 0)
    wait(NCHUNK - 1, 0)
    compute(NCHUNK - 1, 0)

    pltpu.sync_copy(outv, out_hbm.at[pl.ds(base, EPW)])


@jax.jit
def _run(z, src, dst):
    mesh = plsc.VectorSubcoreMesh(
        core_axis_name="c", subcore_axis_name="s",
        num_cores=NC, num_subcores=NS)
    return pl.kernel(
        _sc_body,
        out_type=jax.ShapeDtypeStruct((E,), jnp.float32),
        mesh=mesh,
        compiler_params=pltpu.CompilerParams(needs_layout_passes=False),
        scratch_types=[
            pltpu.VMEM((EPW,), jnp.int32),
            pltpu.VMEM((EPW,), jnp.int32),
            pltpu.VMEM((K, D), jnp.float32),
            pltpu.VMEM((K, D), jnp.float32),
            pltpu.VMEM((K, D), jnp.float32),
            pltpu.VMEM((K, D), jnp.float32),
            pltpu.VMEM((EPW,), jnp.float32),
            pltpu.SemaphoreType.DMA,
            pltpu.SemaphoreType.DMA,
        ],
    )(z, src, dst)


def kernel(z, edge_index):
    ei = edge_index.astype(jnp.int32)
    return _run(z, ei[0], ei[1])
